# Initial kernel scaffold; baseline (speedup 1.0000x reference)
import jax
import jax.numpy as jnp
from jax.experimental import pallas as pl


def kernel(x, pos, batch, W1, b1, W2, b2):
    raise NotImplementedError("write your pallas kernel here")


# speedup vs baseline: 4.6916x; 4.6916x over previous
import functools

import jax
import jax.numpy as jnp
from jax.experimental import pallas as pl
from jax.experimental.pallas import tpu as pltpu

_N = 10000
_M = 5000
_K = 32
_D = 128
_SUB = 8
_LANES = 1280
_NPAD = _SUB * _LANES
_IDXL = 640
_BC = 200
_RB = 512


def _fps_body(px_ref, py_ref, pz_ref, idx_ref, dists_ref):
    si = jax.lax.broadcasted_iota(jnp.int32, (_SUB, _LANES), 0)
    li = jax.lax.broadcasted_iota(jnp.int32, (_SUB, _LANES), 1)
    fj = si * _LANES + li
    dists_ref[...] = jnp.where(fj < _N, jnp.inf, -jnp.inf)
    isi = jax.lax.broadcasted_iota(jnp.int32, (_SUB, _IDXL), 0)
    ili = jax.lax.broadcasted_iota(jnp.int32, (_SUB, _IDXL), 1)
    fpos = isi * _IDXL + ili
    idx_ref[...] = jnp.zeros((_SUB, _IDXL), jnp.int32)
    px = px_ref[...]
    py = py_ref[...]
    pz = pz_ref[...]

    def body(i, cur):
        idx_ref[...] = jnp.where(fpos == i, cur, idx_ref[...])
        eq = fj == cur
        cx = jnp.sum(jnp.where(eq, px, 0.0))
        cy = jnp.sum(jnp.where(eq, py, 0.0))
        cz = jnp.sum(jnp.where(eq, pz, 0.0))
        dx = px - cx
        dy = py - cy
        dz = pz - cz
        d = dx * dx + dy * dy + dz * dz
        nd = jnp.minimum(dists_ref[...], d)
        dists_ref[...] = nd
        mx = jnp.max(nd)
        return jnp.min(jnp.where(nd == mx, fj, _NPAD)).astype(jnp.int32)

    jax.lax.fori_loop(0, _M, body, jnp.int32(0))


def _knn_body(ps_ref, px_ref, py_ref, pz_ref, nbr_ref):
    ps = ps_ref[...]
    cx = ps[:, 0:1]
    cy = ps[:, 1:2]
    cz = ps[:, 2:3]
    dx = cx - px_ref[...]
    dy = cy - py_ref[...]
    dz = cz - pz_ref[...]
    d2 = dx * dx + dy * dy + dz * dz
    li = jax.lax.broadcasted_iota(jnp.int32, (_SUB, _NPAD), 1)
    d2 = jnp.where(li < _N, d2, jnp.inf)
    for k in range(_K):
        m = jnp.min(d2, axis=1, keepdims=True)
        am = jnp.min(jnp.where(d2 == m, li, _NPAD), axis=1, keepdims=True)
        nbr_ref[:, k:k + 1] = am
        d2 = jnp.where(li == am, jnp.inf, d2)


def _u_body(x_ref, p_ref, w1a_ref, w1b_ref, b1_ref, u_ref):
    u_ref[...] = (
        jnp.dot(x_ref[...], w1a_ref[...], preferred_element_type=jnp.float32)
        + jnp.dot(p_ref[...], w1b_ref[...], preferred_element_type=jnp.float32)
        + b1_ref[...]
    )


def _mlp_body(g_ref, ps_ref, w1b_ref, w2_ref, b2_ref, out_ref):
    v = jnp.dot(ps_ref[...], w1b_ref[...], preferred_element_type=jnp.float32)
    acc = jnp.full((_BC, _D), -jnp.inf, dtype=jnp.float32)
    w2 = w2_ref[...]
    b2 = b2_ref[...]
    for k in range(_K):
        h1 = jnp.maximum(g_ref[k] - v, 0.0)
        h2 = jnp.dot(h1, w2, preferred_element_type=jnp.float32) + b2
        acc = jnp.maximum(acc, h2)
    out_ref[...] = jnp.maximum(acc, 0.0)


def kernel(x, pos, batch, W1, b1, W2, b2):
    f32 = jnp.float32
    posp = jnp.pad(pos.astype(f32), ((0, _NPAD - _N), (0, 0)))
    px = posp[:, 0].reshape(_SUB, _LANES)
    py = posp[:, 1].reshape(_SUB, _LANES)
    pz = posp[:, 2].reshape(_SUB, _LANES)

    idx_buf = pl.pallas_call(
        _fps_body,
        out_shape=jax.ShapeDtypeStruct((_SUB, _IDXL), jnp.int32),
        scratch_shapes=[pltpu.VMEM((_SUB, _LANES), f32)],
    )(px, py, pz)
    idx = idx_buf.reshape(-1)[:_M]

    pos_s = jnp.take(pos, idx, axis=0)

    pxr = posp[:, 0].reshape(1, _NPAD)
    pyr = posp[:, 1].reshape(1, _NPAD)
    pzr = posp[:, 2].reshape(1, _NPAD)
    nbr = pl.pallas_call(
        _knn_body,
        grid=(_M // _SUB,),
        in_specs=[
            pl.BlockSpec((_SUB, 3), lambda b: (b, 0)),
            pl.BlockSpec((1, _NPAD), lambda b: (0, 0)),
            pl.BlockSpec((1, _NPAD), lambda b: (0, 0)),
            pl.BlockSpec((1, _NPAD), lambda b: (0, 0)),
        ],
        out_specs=pl.BlockSpec((_SUB, _K), lambda b: (b, 0)),
        out_shape=jax.ShapeDtypeStruct((_M, _K), jnp.int32),
    )(pos_s, pxr, pyr, pzr)

    W1a = W1[:_D, :]
    W1b = W1[_D:, :]
    b1r = b1.reshape(1, _D)
    xp = jnp.pad(x.astype(f32), ((0, _NPAD - _N), (0, 0)))
    u = pl.pallas_call(
        _u_body,
        grid=(_NPAD // _RB,),
        in_specs=[
            pl.BlockSpec((_RB, _D), lambda b: (b, 0)),
            pl.BlockSpec((_RB, 3), lambda b: (b, 0)),
            pl.BlockSpec((_D, _D), lambda b: (0, 0)),
            pl.BlockSpec((3, _D), lambda b: (0, 0)),
            pl.BlockSpec((1, _D), lambda b: (0, 0)),
        ],
        out_specs=pl.BlockSpec((_RB, _D), lambda b: (b, 0)),
        out_shape=jax.ShapeDtypeStruct((_NPAD, _D), f32),
    )(xp, posp, W1a, W1b, b1r)

    col = nbr.T.reshape(-1)
    g = jnp.take(u, col, axis=0).reshape(_K, _M, _D)

    b2r = b2.reshape(1, _D)
    out = pl.pallas_call(
        _mlp_body,
        grid=(_M // _BC,),
        in_specs=[
            pl.BlockSpec((_K, _BC, _D), lambda b: (0, b, 0)),
            pl.BlockSpec((_BC, 3), lambda b: (b, 0)),
            pl.BlockSpec((3, _D), lambda b: (0, 0)),
            pl.BlockSpec((_D, _D), lambda b: (0, 0)),
            pl.BlockSpec((1, _D), lambda b: (0, 0)),
        ],
        out_specs=pl.BlockSpec((_BC, _D), lambda b: (b, 0)),
        out_shape=jax.ShapeDtypeStruct((_M, _D), f32),
    )(g, pos_s, W1b, W2, b2r)

    return (out, pos_s, jnp.take(batch, idx, axis=0))
